# Initial kernel scaffold; baseline (speedup 1.0000x reference)
import jax
import jax.numpy as jnp
from jax.experimental import pallas as pl


def kernel(p1, p2):
    raise NotImplementedError("write your pallas kernel here")


# speedup vs baseline: 23.8113x; 23.8113x over previous
import functools

import jax
import jax.numpy as jnp
from jax import lax
from jax.experimental import pallas as pl

K = 8
BQ = 256
CK = 2048

_INF = float("inf")
_IBIG = 2**30


def _select_topk(vals, idx, n_rounds):
    out_v = []
    out_i = []
    for _ in range(n_rounds):
        m = jnp.min(vals, axis=1, keepdims=True)
        sel = jnp.where(vals == m, idx, _IBIG)
        j = jnp.min(sel, axis=1, keepdims=True)
        vals = jnp.where(sel == j, _INF, vals)
        out_v.append(m)
        out_i.append(j)
    return jnp.concatenate(out_v, axis=1), jnp.concatenate(out_i, axis=1)


def _knn_body(p1_ref, p2_ref, dist_ref, idx_ref):
    p1b = p1_ref[0]
    p1_sq = jnp.sum(p1b * p1b, axis=1, keepdims=True)
    lane = lax.broadcasted_iota(jnp.int32, (BQ, CK), 1)

    n2 = p2_ref.shape[1]
    n_chunks = n2 // CK

    def chunk_step(c, carry):
        run_v, run_i = carry
        p2c = p2_ref[0, pl.ds(c * CK, CK), :]
        inner = lax.dot_general(
            p1b, p2c,
            dimension_numbers=(((1,), (1,)), ((), ())),
            preferred_element_type=jnp.float32,
        )
        p2_sq = jnp.sum(p2c * p2c, axis=1)[None, :]
        d = p1_sq + p2_sq - 2.0 * inner
        gidx = c * CK + lane
        cv, ci = _select_topk(d, gidx, K)
        ev = jnp.concatenate([run_v, cv], axis=1)
        ei = jnp.concatenate([run_i, ci], axis=1)
        return _select_topk(ev, ei, K)

    init = (jnp.full((BQ, K), _INF, jnp.float32),
            jnp.full((BQ, K), _IBIG, jnp.int32))
    run_v, run_i = lax.fori_loop(0, n_chunks, chunk_step, init)
    dist_ref[0] = run_v
    idx_ref[0] = run_i


def _knn(p1, p2):
    b, n1, d = p1.shape
    _, n2, _ = p2.shape
    grid = (b, n1 // BQ)
    return pl.pallas_call(
        _knn_body,
        grid=grid,
        in_specs=[
            pl.BlockSpec((1, BQ, d), lambda i, j: (i, j, 0)),
            pl.BlockSpec((1, n2, d), lambda i, j: (i, 0, 0)),
        ],
        out_specs=[
            pl.BlockSpec((1, BQ, K), lambda i, j: (i, j, 0)),
            pl.BlockSpec((1, BQ, K), lambda i, j: (i, j, 0)),
        ],
        out_shape=[
            jax.ShapeDtypeStruct((b, n1, K), jnp.float32),
            jax.ShapeDtypeStruct((b, n1, K), jnp.int32),
        ],
    )(p1, p2)


@jax.jit
def kernel(p1, p2):
    dists, idx = _knn(p1, p2)
    return dists, idx
